# baseline (speedup 1.0000x reference)
import functools

import jax
import jax.numpy as jnp
from jax import lax
from jax.experimental import pallas as pl
from jax.experimental.pallas import tpu as pltpu
from jax.experimental.pallas import tpu_sc as plsc

N_NODES = 10000
N_EDGES = 320000
IN_CH = 128
HID = 64
OUT_CH = 40

CHUNK = 256
NCH_D = 40
NCH_P = 80
E_PAD = 32 * NCH_D * CHUNK
N_PAD = 10240
RPT = N_PAD // 16

_MESH = plsc.VectorSubcoreMesh(core_axis_name="c", subcore_axis_name="s")
_SC_PARAMS = pltpu.CompilerParams(use_tc_tiling_on_sc=False)


def _deg_body(col_hbm, ones_hbm, zero_hbm, out_hbm, col_v, ones_v, degsh):
    c = lax.axis_index("c")
    s = lax.axis_index("s")
    wid = s * 2 + c
    pltpu.sync_copy(zero_hbm, degsh.at[pl.ds(s * RPT, RPT)])
    pltpu.sync_copy(col_hbm.at[wid], col_v)
    pltpu.sync_copy(ones_hbm, ones_v)
    plsc.subcore_barrier()

    def body(g, carry):
        pltpu.sync_copy(ones_v, degsh.at[col_v.at[g]], add=True)
        return carry

    lax.fori_loop(0, NCH_D, body, 0)
    plsc.subcore_barrier()
    pltpu.sync_copy(degsh.at[pl.ds(s * RPT, RPT)],
                    out_hbm.at[c, pl.ds(s * RPT, RPT)])


_sc_deg = functools.partial(
    pl.kernel,
    _deg_body,
    out_type=jax.ShapeDtypeStruct((2, N_PAD), jnp.float32),
    mesh=_MESH,
    compiler_params=_SC_PARAMS,
    scratch_types=[
        pltpu.VMEM((NCH_D, CHUNK), jnp.int32),
        pltpu.VMEM((CHUNK,), jnp.float32),
        pltpu.VMEM_SHARED((N_PAD,), jnp.float32),
    ],
)()


def _prop_body(src_hbm, row_hbm, col_hbm, zero_hbm, out_l, out_r,
               row_v, col_v, b0, b1, zsh, gs0, gs1, ss0, ss1):
    c = lax.axis_index("c")
    s = lax.axis_index("s")
    pltpu.sync_copy(zero_hbm, zsh.at[pl.ds(s * RPT, RPT)])
    pltpu.sync_copy(row_hbm.at[s], row_v)
    pltpu.sync_copy(col_hbm.at[s], col_v)
    plsc.subcore_barrier()

    src = src_hbm.at[c]
    bufs = (b0, b1)
    gsems = (gs0, gs1)
    ssems = (ss0, ss1)

    def gather(g, b):
        return pltpu.make_async_copy(src.at[row_v.at[g]], bufs[b], gsems[b])

    def scatter(g, b):
        return pltpu.make_async_copy(bufs[b], zsh.at[col_v.at[g]], ssems[b])

    gather(0, 0).start()

    def body(g2, carry):
        for b in range(2):
            g = 2 * g2 + b
            gather(g, b).wait()
            scatter(g, b).start(add=True)
            nb = 1 - b

            @pl.when(g >= 1)
            def _():
                scatter(g, nb).wait()

            @pl.when(g + 1 < NCH_P)
            def _():
                gather(g + 1, nb).start()

        return carry

    lax.fori_loop(0, NCH_P // 2, body, 0)
    scatter(NCH_P - 1, (NCH_P - 1) % 2).wait()
    plsc.subcore_barrier()

    @pl.when(c == 0)
    def _():
        pltpu.sync_copy(zsh.at[pl.ds(s * RPT, RPT)],
                        out_l.at[pl.ds(s * RPT, RPT)])

    @pl.when(c == 1)
    def _():
        pltpu.sync_copy(zsh.at[pl.ds(s * RPT, RPT)],
                        out_r.at[pl.ds(s * RPT, RPT)])


def _make_prop(width):
    return functools.partial(
        pl.kernel,
        _prop_body,
        out_type=[jax.ShapeDtypeStruct((N_PAD, width), jnp.float32),
                  jax.ShapeDtypeStruct((N_PAD, width), jnp.float32)],
        mesh=_MESH,
        compiler_params=_SC_PARAMS,
        scratch_types=(
            [pltpu.VMEM((NCH_P, CHUNK), jnp.int32)] * 2
            + [pltpu.VMEM((CHUNK, width), jnp.float32)] * 2
            + [pltpu.VMEM_SHARED((N_PAD, width), jnp.float32)]
            + [pltpu.SemaphoreType.DMA] * 4
        ),
    )()


_sc_prop64 = _make_prop(HID)
_sc_prop32 = _make_prop(HID // 2)


def _tc1a_body(xp_ref, w012_ref, t_ref):
    t_ref[...] = jnp.dot(xp_ref[...], w012_ref[...],
                         preferred_element_type=jnp.float32)


def _tc1b_body(t_ref, degp_ref, ys_ref, dis_ref):
    deg = degp_ref[0] + degp_ref[1] + 1.0
    dis = lax.rsqrt(deg)
    ys_ref[0] = t_ref[:, HID:2 * HID] * dis
    ys_ref[1] = t_ref[:, 2 * HID:] * dis
    dis_ref[...] = dis


def _tc2_body(zr_ref, ys_ref, dis_ref, ws_ref):
    dis = dis_ref[...]
    wsfull = (zr_ref[...] + ys_ref[1]) * (dis * dis)
    ws_ref[0] = wsfull[:, :HID // 2]
    ws_ref[1] = wsfull[:, HID // 2:]


def _tc3_body(t_ref, zl_ref, z2l_ref, z2r_ref, ws_ref, ys_ref, dis_ref,
              b0_ref, b1_ref, b2_ref, wlin_ref, blin_ref, out_ref):
    dis = dis_ref[...]
    out0 = t_ref[:, :HID] + b0_ref[...]
    out1 = (zl_ref[...] + ys_ref[0]) * dis + b1_ref[...]
    z2raw = jnp.concatenate([z2l_ref[...], z2r_ref[...]], axis=-1)
    ws = jnp.concatenate([ws_ref[0], ws_ref[1]], axis=-1)
    out2 = (z2raw + ws) * dis + b2_ref[...]
    h = jnp.concatenate([out0, out1, out2], axis=-1)
    h = jnp.maximum(h, 0.0)
    out_ref[...] = jnp.dot(h, wlin_ref[...],
                           preferred_element_type=jnp.float32) + blin_ref[...]


def kernel(x, edge_index, W0, b0, W1, b1, W2, b2, Wlin, blin):
    f32 = jnp.float32
    xp = jnp.pad(x, ((0, N_PAD - N_NODES), (0, 0)))
    row = edge_index[0]
    col = edge_index[1]
    pad_e = E_PAD - N_EDGES
    ar = jnp.arange(pad_e, dtype=jnp.int32)
    pad_row = (ar * 9973) % N_NODES
    pad_col = N_NODES + ar % (N_PAD - N_NODES)
    rowp = jnp.concatenate([row, pad_row])
    colp = jnp.concatenate([col, pad_col])
    row16 = rowp.reshape(16, NCH_P, CHUNK)
    col16 = colp.reshape(16, NCH_P, CHUNK)
    col32 = colp.reshape(32, NCH_D, CHUNK)

    ones_c = jnp.ones((CHUNK,), f32)
    z1d = jnp.zeros((RPT,), f32)
    z64 = jnp.zeros((RPT, HID), f32)
    z32 = jnp.zeros((RPT, HID // 2), f32)

    degp = _sc_deg(col32, ones_c, z1d)

    w012 = jnp.concatenate([W0, W1, W2], axis=1)
    t = pl.pallas_call(
        _tc1a_body,
        out_shape=jax.ShapeDtypeStruct((N_PAD, 3 * HID), f32),
    )(xp, w012)

    ys, dis = pl.pallas_call(
        _tc1b_body,
        out_shape=[
            jax.ShapeDtypeStruct((2, N_PAD, HID), f32),
            jax.ShapeDtypeStruct((N_PAD, 1), f32),
        ],
    )(t, degp.reshape(2, N_PAD, 1))

    zl, zr = _sc_prop64(ys, row16, col16, z64)

    ws = pl.pallas_call(
        _tc2_body,
        out_shape=jax.ShapeDtypeStruct((2, N_PAD, HID // 2), f32),
    )(zr, ys, dis)

    z2l, z2r = _sc_prop32(ws, row16, col16, z32)

    BM = N_PAD // 8
    row_blk = lambda w: pl.BlockSpec((BM, w), lambda i: (i, 0))
    pair_blk = lambda w: pl.BlockSpec((2, BM, w), lambda i: (0, i, 0))
    full = lambda a: pl.BlockSpec(a.shape, lambda i: (0,) * a.ndim)
    b0r = b0.reshape(1, HID)
    b1r = b1.reshape(1, HID)
    b2r = b2.reshape(1, HID)
    blinr = blin.reshape(1, OUT_CH)
    out = pl.pallas_call(
        _tc3_body,
        grid=(8,),
        in_specs=[
            row_blk(3 * HID), row_blk(HID), row_blk(HID // 2),
            row_blk(HID // 2), pair_blk(HID // 2), pair_blk(HID),
            row_blk(1), full(b0r), full(b1r), full(b2r), full(Wlin),
            full(blinr),
        ],
        out_specs=row_blk(OUT_CH),
        out_shape=jax.ShapeDtypeStruct((N_PAD, OUT_CH), f32),
    )(t, zl, z2l, z2r, ws, ys, dis, b0r, b1r, b2r, Wlin, blinr)

    return out[:N_NODES]


# speedup vs baseline: 1.1172x; 1.1172x over previous
import functools

import jax
import jax.numpy as jnp
from jax import lax
from jax.experimental import pallas as pl
from jax.experimental.pallas import tpu as pltpu
from jax.experimental.pallas import tpu_sc as plsc

N_NODES = 10000
N_EDGES = 320000
IN_CH = 128
HID = 64
OUT_CH = 40

CHUNK = 256
NCH_D = 40
NCH_P = 80
E_PAD = 32 * NCH_D * CHUNK
N_PAD = 10240
RPT = N_PAD // 16

_MESH = plsc.VectorSubcoreMesh(core_axis_name="c", subcore_axis_name="s")
_SC_PARAMS = pltpu.CompilerParams(use_tc_tiling_on_sc=False)


def _deg_body(col_hbm, ones_hbm, zero_hbm, out_hbm, col_v, ones_v, degsh):
    c = lax.axis_index("c")
    s = lax.axis_index("s")
    wid = s * 2 + c
    pltpu.sync_copy(zero_hbm, degsh.at[pl.ds(s * RPT, RPT)])
    pltpu.sync_copy(col_hbm.at[wid], col_v)
    pltpu.sync_copy(ones_hbm, ones_v)
    plsc.subcore_barrier()

    def body(g, carry):
        pltpu.sync_copy(ones_v, degsh.at[col_v.at[g]], add=True)
        return carry

    lax.fori_loop(0, NCH_D, body, 0)
    plsc.subcore_barrier()
    pltpu.sync_copy(degsh.at[pl.ds(s * RPT, RPT)],
                    out_hbm.at[c, pl.ds(s * RPT, RPT)])


_sc_deg = functools.partial(
    pl.kernel,
    _deg_body,
    out_type=jax.ShapeDtypeStruct((2, N_PAD), jnp.float32),
    mesh=_MESH,
    compiler_params=_SC_PARAMS,
    scratch_types=[
        pltpu.VMEM((NCH_D, CHUNK), jnp.int32),
        pltpu.VMEM((CHUNK,), jnp.float32),
        pltpu.VMEM_SHARED((N_PAD,), jnp.float32),
    ],
)()


CPP = 40


def _prop_body(src_hbm, row_hbm, col_hbm, zero_hbm, out_l, out_r,
               row_v, col_v, b0, b1, b2, b3, zsh,
               gs0, gs1, gs2, gs3, ss0, ss1, ss2, ss3):
    c = lax.axis_index("c")
    s = lax.axis_index("s")
    pltpu.sync_copy(zero_hbm, zsh.at[pl.ds(s * RPT, RPT)])
    plsc.subcore_barrier()

    src = src_hbm.at[c]
    bufs = (b0, b1, b2, b3)
    gsems = (gs0, gs1, gs2, gs3)
    ssems = (ss0, ss1, ss2, ss3)

    def gather(g, b):
        return pltpu.make_async_copy(src.at[row_v.at[g]], bufs[b], gsems[b])

    def scatter(g, b):
        return pltpu.make_async_copy(bufs[b], zsh.at[col_v.at[g]], ssems[b])

    for ph in range(NCH_P // CPP):
        pltpu.sync_copy(row_hbm.at[s, pl.ds(ph * CPP, CPP)], row_v)
        pltpu.sync_copy(col_hbm.at[s, pl.ds(ph * CPP, CPP)], col_v)
        gather(0, 0).start()
        gather(1, 1).start()

        def body(g4, carry):
            for b in range(4):
                g = 4 * g4 + b
                gather(g, b).wait()
                scatter(g, b).start(add=True)
                nb = (b + 2) % 4

                @pl.when(g >= 2)
                def _():
                    scatter(g, nb).wait()

                @pl.when(g + 2 < CPP)
                def _():
                    gather(g + 2, nb).start()

            return carry

        lax.fori_loop(0, CPP // 4, body, 0)
        scatter(CPP - 2, (CPP - 2) % 4).wait()
        scatter(CPP - 1, (CPP - 1) % 4).wait()

    plsc.subcore_barrier()

    @pl.when(c == 0)
    def _():
        pltpu.sync_copy(zsh.at[pl.ds(s * RPT, RPT)],
                        out_l.at[pl.ds(s * RPT, RPT)])

    @pl.when(c == 1)
    def _():
        pltpu.sync_copy(zsh.at[pl.ds(s * RPT, RPT)],
                        out_r.at[pl.ds(s * RPT, RPT)])


def _make_prop(width):
    return functools.partial(
        pl.kernel,
        _prop_body,
        out_type=[jax.ShapeDtypeStruct((N_PAD, width), jnp.float32),
                  jax.ShapeDtypeStruct((N_PAD, width), jnp.float32)],
        mesh=_MESH,
        compiler_params=_SC_PARAMS,
        scratch_types=(
            [pltpu.VMEM((CPP, CHUNK), jnp.int32)] * 2
            + [pltpu.VMEM((CHUNK, width), jnp.float32)] * 4
            + [pltpu.VMEM_SHARED((N_PAD, width), jnp.float32)]
            + [pltpu.SemaphoreType.DMA] * 8
        ),
    )()


_sc_prop64 = _make_prop(HID)
_sc_prop32 = _make_prop(HID // 2)


def _tc1a_body(xp_ref, w012_ref, t_ref):
    t_ref[...] = jnp.dot(xp_ref[...], w012_ref[...],
                         preferred_element_type=jnp.float32)


def _tc1b_body(t_ref, degp_ref, ys_ref, dis_ref):
    deg = degp_ref[0] + degp_ref[1] + 1.0
    dis = lax.rsqrt(deg)
    ys_ref[0] = t_ref[:, HID:2 * HID] * dis
    ys_ref[1] = t_ref[:, 2 * HID:] * dis
    dis_ref[...] = dis


def _tc2_body(zr_ref, ys_ref, dis_ref, ws_ref):
    dis = dis_ref[...]
    wsfull = (zr_ref[...] + ys_ref[1]) * (dis * dis)
    ws_ref[0] = wsfull[:, :HID // 2]
    ws_ref[1] = wsfull[:, HID // 2:]


def _tc3_body(t_ref, zl_ref, z2l_ref, z2r_ref, ws_ref, ys_ref, dis_ref,
              b0_ref, b1_ref, b2_ref, wlin_ref, blin_ref, out_ref):
    dis = dis_ref[...]
    out0 = t_ref[:, :HID] + b0_ref[...]
    out1 = (zl_ref[...] + ys_ref[0]) * dis + b1_ref[...]
    z2raw = jnp.concatenate([z2l_ref[...], z2r_ref[...]], axis=-1)
    ws = jnp.concatenate([ws_ref[0], ws_ref[1]], axis=-1)
    out2 = (z2raw + ws) * dis + b2_ref[...]
    h = jnp.concatenate([out0, out1, out2], axis=-1)
    h = jnp.maximum(h, 0.0)
    out_ref[...] = jnp.dot(h, wlin_ref[...],
                           preferred_element_type=jnp.float32) + blin_ref[...]


def kernel(x, edge_index, W0, b0, W1, b1, W2, b2, Wlin, blin):
    f32 = jnp.float32
    xp = jnp.pad(x, ((0, N_PAD - N_NODES), (0, 0)))
    row = edge_index[0]
    col = edge_index[1]
    pad_e = E_PAD - N_EDGES
    ar = jnp.arange(pad_e, dtype=jnp.int32)
    pad_row = (ar * 9973) % N_NODES
    pad_col = N_NODES + ar % (N_PAD - N_NODES)
    rowp = jnp.concatenate([row, pad_row])
    colp = jnp.concatenate([col, pad_col])
    row16 = rowp.reshape(16, NCH_P, CHUNK)
    col16 = colp.reshape(16, NCH_P, CHUNK)
    col32 = colp.reshape(32, NCH_D, CHUNK)

    ones_c = jnp.ones((CHUNK,), f32)
    z1d = jnp.zeros((RPT,), f32)
    z64 = jnp.zeros((RPT, HID), f32)
    z32 = jnp.zeros((RPT, HID // 2), f32)

    degp = _sc_deg(col32, ones_c, z1d)

    w012 = jnp.concatenate([W0, W1, W2], axis=1)
    t = pl.pallas_call(
        _tc1a_body,
        out_shape=jax.ShapeDtypeStruct((N_PAD, 3 * HID), f32),
    )(xp, w012)

    ys, dis = pl.pallas_call(
        _tc1b_body,
        out_shape=[
            jax.ShapeDtypeStruct((2, N_PAD, HID), f32),
            jax.ShapeDtypeStruct((N_PAD, 1), f32),
        ],
    )(t, degp.reshape(2, N_PAD, 1))

    zl, zr = _sc_prop64(ys, row16, col16, z64)

    ws = pl.pallas_call(
        _tc2_body,
        out_shape=jax.ShapeDtypeStruct((2, N_PAD, HID // 2), f32),
    )(zr, ys, dis)

    z2l, z2r = _sc_prop32(ws, row16, col16, z32)

    BM = N_PAD // 8
    row_blk = lambda w: pl.BlockSpec((BM, w), lambda i: (i, 0))
    pair_blk = lambda w: pl.BlockSpec((2, BM, w), lambda i: (0, i, 0))
    full = lambda a: pl.BlockSpec(a.shape, lambda i: (0,) * a.ndim)
    b0r = b0.reshape(1, HID)
    b1r = b1.reshape(1, HID)
    b2r = b2.reshape(1, HID)
    blinr = blin.reshape(1, OUT_CH)
    out = pl.pallas_call(
        _tc3_body,
        grid=(8,),
        in_specs=[
            row_blk(3 * HID), row_blk(HID), row_blk(HID // 2),
            row_blk(HID // 2), pair_blk(HID // 2), pair_blk(HID),
            row_blk(1), full(b0r), full(b1r), full(b2r), full(Wlin),
            full(blinr),
        ],
        out_specs=row_blk(OUT_CH),
        out_shape=jax.ShapeDtypeStruct((N_PAD, OUT_CH), f32),
    )(t, zl, z2l, z2r, ws, ys, dis, b0r, b1r, b2r, Wlin, blinr)

    return out[:N_NODES]
